# baseline (speedup 1.0000x reference)
import functools

import jax
import jax.numpy as jnp
from jax import lax
from jax.experimental import pallas as pl
from jax.experimental.pallas import tpu as pltpu
from jax.experimental.pallas import tpu_sc as plsc

HIDDEN = 64
PAIR = 64
T2_BLK = 8192
N_TABLE = 1000000


def _t2_body(t_ref, w_ref, b_ref, o_ref):
    x = jnp.maximum(t_ref[...], 0.0)
    y = lax.dot_general(
        x.T, w_ref[...], (((1,), (1,)), ((), ())),
        preferred_element_type=jnp.float32) + b_ref[...]
    o_ref[:, :HIDDEN] = y


def _make_gather(B, n_workers, b_per_w, n_chunks):
    mesh = plsc.VectorSubcoreMesh(core_axis_name="c", subcore_axis_name="s")
    n_pairs = n_chunks // 2
    pairs_per_w = b_per_w // 2

    @functools.partial(
        pl.kernel,
        mesh=mesh,
        out_type=jax.ShapeDtypeStruct((B // 2, 2 * HIDDEN), jnp.float32),
        scratch_types=[
            pltpu.VMEM((n_chunks, 2 * PAIR), jnp.int32),
            pltpu.VMEM((2, 2, PAIR, 2 * HIDDEN), jnp.float32),
            pltpu.SemaphoreType.DMA,
            pltpu.SemaphoreType.DMA,
        ],
        compiler_params=pltpu.CompilerParams(use_tc_tiling_on_sc=False),
    )
    def gather_k(idx_hbm, table2_hbm, out_hbm, idx_v, rows_v, sem0, sem1):
        nc = lax.axis_size("c")
        wid = lax.axis_index("s") * nc + lax.axis_index("c")
        pair_base = wid * pairs_per_w

        pltpu.sync_copy(idx_hbm.at[wid], idx_v)

        def copies(c, buf, sem):
            for g in range(2):
                src = table2_hbm.at[idx_v.at[c, pl.ds(PAIR * g, PAIR)]]
                dst = rows_v.at[buf, g]
                yield src, dst, sem

        def start(c, buf, sem):
            for src, dst, s in copies(c, buf, sem):
                pltpu.async_copy(src, dst, s)

        def wait(c, buf, sem):
            for src, dst, s in copies(c, buf, sem):
                pltpu.make_async_copy(src, dst, s).wait()

        def store(c, buf):
            for g in range(2):
                pltpu.sync_copy(
                    rows_v.at[buf, g, :, pl.ds(0, HIDDEN)],
                    out_hbm.at[pl.ds(pair_base + c * PAIR, PAIR),
                               pl.ds(HIDDEN * g, HIDDEN)])

        start(0, 0, sem0)

        def body(i, carry):
            c0 = 2 * i
            start(c0 + 1, 1, sem1)
            wait(c0, 0, sem0)
            store(c0, 0)
            start(c0 + 2, 0, sem0)
            wait(c0 + 1, 1, sem1)
            store(c0 + 1, 1)
            return carry

        lax.fori_loop(0, n_pairs - 1, body, 0)

        c0 = n_chunks - 2
        start(c0 + 1, 1, sem1)
        wait(c0, 0, sem0)
        store(c0, 0)
        wait(c0 + 1, 1, sem1)
        store(c0 + 1, 1)

    return gather_k


def kernel(inputs, table, W, b):
    batch, seq = inputs.shape
    B = batch * seq
    info = plsc.get_sparse_core_info()
    n_workers = info.num_cores * info.num_subcores
    b_per_w = B // n_workers
    n_chunks = b_per_w // (2 * PAIR)

    f3 = inputs.reshape(n_workers, n_chunks, PAIR, 2)
    idx2 = jnp.concatenate([f3[..., 0], f3[..., 1]], axis=-1)

    table2 = pl.pallas_call(
        _t2_body,
        grid=((N_TABLE + T2_BLK - 1) // T2_BLK,),
        in_specs=[
            pl.BlockSpec((HIDDEN, T2_BLK), lambda i: (0, i)),
            pl.BlockSpec((HIDDEN, HIDDEN), lambda i: (0, 0)),
            pl.BlockSpec((1, HIDDEN), lambda i: (0, 0)),
        ],
        out_specs=pl.BlockSpec((T2_BLK, 2 * HIDDEN), lambda i: (i, 0)),
        out_shape=jax.ShapeDtypeStruct((N_TABLE, 2 * HIDDEN), jnp.float32),
    )(table.T, W, b.reshape(1, HIDDEN))

    g128 = _make_gather(B, n_workers, b_per_w, n_chunks)(idx2, table2)

    return g128.reshape(batch, seq, HIDDEN)


# speedup vs baseline: 1.6852x; 1.6852x over previous
import functools

import jax
import jax.numpy as jnp
from jax import lax
from jax.experimental import pallas as pl
from jax.experimental.pallas import tpu as pltpu
from jax.experimental.pallas import tpu_sc as plsc

HIDDEN = 64
PAIR = 64
T2_BLK = 8192
UNP_BLK = 4096
N_TABLE = 1000000


def _unpack_body(x_ref, o_ref):
    x = x_ref[...]
    o_ref[0] = x[:, :HIDDEN]
    o_ref[1] = x[:, HIDDEN:]


def _t2_body(t_ref, w_ref, b_ref, o_ref):
    x = jnp.maximum(t_ref[...], 0.0)
    y = lax.dot_general(
        x.T, w_ref[...], (((1,), (1,)), ((), ())),
        preferred_element_type=jnp.float32) + b_ref[...]
    o_ref[:, :HIDDEN] = y


def _make_gather(B, n_workers, b_per_w, n_chunks):
    mesh = plsc.VectorSubcoreMesh(core_axis_name="c", subcore_axis_name="s")
    n_pairs = n_chunks // 2
    pairs_per_w = b_per_w // 2

    @functools.partial(
        pl.kernel,
        mesh=mesh,
        out_type=jax.ShapeDtypeStruct((B // 2, 2 * HIDDEN), jnp.float32),
        scratch_types=[
            pltpu.VMEM((n_chunks, 2 * PAIR), jnp.int32),
            pltpu.VMEM((2, 2, PAIR, 2 * HIDDEN), jnp.float32),
            pltpu.SemaphoreType.DMA,
            pltpu.SemaphoreType.DMA,
        ],
        compiler_params=pltpu.CompilerParams(use_tc_tiling_on_sc=False),
    )
    def gather_k(idx_hbm, table2_hbm, out_hbm, idx_v, rows_v, sem0, sem1):
        nc = lax.axis_size("c")
        wid = lax.axis_index("s") * nc + lax.axis_index("c")
        pair_base = wid * pairs_per_w

        pltpu.sync_copy(idx_hbm.at[wid], idx_v)

        def copies(c, buf, sem):
            for g in range(2):
                src = table2_hbm.at[idx_v.at[c, pl.ds(PAIR * g, PAIR)]]
                dst = rows_v.at[buf, g]
                yield src, dst, sem

        def start(c, buf, sem):
            for src, dst, s in copies(c, buf, sem):
                pltpu.async_copy(src, dst, s)

        def wait(c, buf, sem):
            for src, dst, s in copies(c, buf, sem):
                pltpu.make_async_copy(src, dst, s).wait()

        def store(c, buf):
            for g in range(2):
                pltpu.sync_copy(
                    rows_v.at[buf, g, :, pl.ds(0, HIDDEN)],
                    out_hbm.at[pl.ds(pair_base + c * PAIR, PAIR),
                               pl.ds(HIDDEN * g, HIDDEN)])

        start(0, 0, sem0)

        def body(i, carry):
            c0 = 2 * i
            start(c0 + 1, 1, sem1)
            wait(c0, 0, sem0)
            store(c0, 0)
            start(c0 + 2, 0, sem0)
            wait(c0 + 1, 1, sem1)
            store(c0 + 1, 1)
            return carry

        lax.fori_loop(0, n_pairs - 1, body, 0)

        c0 = n_chunks - 2
        start(c0 + 1, 1, sem1)
        wait(c0, 0, sem0)
        store(c0, 0)
        wait(c0 + 1, 1, sem1)
        store(c0 + 1, 1)

    return gather_k


def kernel(inputs, table, W, b):
    batch, seq = inputs.shape
    B = batch * seq
    info = plsc.get_sparse_core_info()
    n_workers = info.num_cores * info.num_subcores
    b_per_w = B // n_workers
    n_chunks = b_per_w // (2 * PAIR)

    flat = inputs.reshape(B)
    ia = flat[:B // 2].reshape(n_workers, n_chunks, PAIR)
    ib = flat[B // 2:].reshape(n_workers, n_chunks, PAIR)
    idx2 = jnp.concatenate([ia, ib], axis=-1)

    table2 = pl.pallas_call(
        _t2_body,
        grid=((N_TABLE + T2_BLK - 1) // T2_BLK,),
        in_specs=[
            pl.BlockSpec((HIDDEN, T2_BLK), lambda i: (0, i)),
            pl.BlockSpec((HIDDEN, HIDDEN), lambda i: (0, 0)),
            pl.BlockSpec((1, HIDDEN), lambda i: (0, 0)),
        ],
        out_specs=pl.BlockSpec((T2_BLK, 2 * HIDDEN), lambda i: (i, 0)),
        out_shape=jax.ShapeDtypeStruct((N_TABLE, 2 * HIDDEN), jnp.float32),
    )(table.T, W, b.reshape(1, HIDDEN))

    g128 = _make_gather(B, n_workers, b_per_w, n_chunks)(idx2, table2)

    out = pl.pallas_call(
        _unpack_body,
        grid=(B // (2 * UNP_BLK),),
        in_specs=[pl.BlockSpec((UNP_BLK, 2 * HIDDEN), lambda i: (i, 0))],
        out_specs=pl.BlockSpec((2, UNP_BLK, HIDDEN), lambda i: (0, i, 0)),
        out_shape=jax.ShapeDtypeStruct((2, B // 2, HIDDEN), jnp.float32),
    )(g128)

    return out.reshape(batch, seq, HIDDEN)
